# Initial kernel scaffold; baseline (speedup 1.0000x reference)
import jax
import jax.numpy as jnp
from jax.experimental import pallas as pl


def kernel(node_ids, edge_index, edge_weight, entity_table, W0, W1):
    raise NotImplementedError("write your pallas kernel here")


# speedup vs baseline: 2.6114x; 2.6114x over previous
import functools

import jax
import jax.numpy as jnp
from jax import lax
from jax.experimental import pallas as pl
from jax.experimental.pallas import tpu as pltpu
from jax.experimental.pallas import tpu_sc as plsc

N_NODES = 10000
D_IN = 128
N_EDGES = 320000

N_TILES = 32
CHUNK = 128
N_CHUNKS = 80
E_PAD = N_TILES * N_CHUNKS * CHUNK
N_ACC = 10240
ROWS_PER_TILE = N_ACC // 16


def _seg_body(h_hbm, src_hbm, dst_hbm, w_hbm, zeros_hbm, out_hbm,
              src_v, dst_v, w_v, rows_v, acc_sh, sem):
    cid = lax.axis_index("c")
    sid = lax.axis_index("s")
    wid = sid * 2 + cid

    pltpu.sync_copy(src_hbm.at[wid], src_v)
    pltpu.sync_copy(dst_hbm.at[wid], dst_v)
    pltpu.sync_copy(w_hbm.at[wid], w_v)
    pltpu.sync_copy(zeros_hbm, acc_sh.at[pl.ds(sid * ROWS_PER_TILE, ROWS_PER_TILE)])
    plsc.subcore_barrier()

    def chunk_body(c, carry):
        pltpu.async_copy(h_hbm.at[src_v.at[c]], rows_v, sem).wait()

        def edge_body(e, carry2):
            w_e = plsc.load_gather(
                w_v, [jnp.full((16,), c, jnp.int32), jnp.full((16,), e, jnp.int32)])
            for j in range(D_IN // 16):
                sl = pl.ds(j * 16, 16)
                rows_v[e, sl] = rows_v[e, sl] * w_e
            return carry2

        lax.fori_loop(0, CHUNK, edge_body, 0, unroll=2)

        pltpu.sync_copy(rows_v, acc_sh.at[dst_v.at[c]], add=True)
        return carry

    lax.fori_loop(0, N_CHUNKS, chunk_body, 0)
    plsc.subcore_barrier()

    sl = pl.ds(sid * ROWS_PER_TILE, ROWS_PER_TILE)
    pltpu.sync_copy(acc_sh.at[sl], out_hbm.at[cid, sl])


_seg_call = functools.partial(
    pl.kernel,
    out_type=jax.ShapeDtypeStruct((2, N_ACC, D_IN), jnp.float32),
    mesh=plsc.VectorSubcoreMesh(core_axis_name="c", subcore_axis_name="s"),
    compiler_params=pltpu.CompilerParams(needs_layout_passes=False),
    scratch_types=[
        pltpu.VMEM((N_CHUNKS, CHUNK), jnp.int32),
        pltpu.VMEM((N_CHUNKS, CHUNK), jnp.int32),
        pltpu.VMEM((N_CHUNKS, CHUNK), jnp.float32),
        pltpu.VMEM((CHUNK, D_IN), jnp.float32),
        pltpu.VMEM_SHARED((N_ACC, D_IN), jnp.float32),
        pltpu.SemaphoreType.DMA,
    ],
)(_seg_body)


def _dense_body(part_ref, h_ref, w_ref, h_out_ref, n_out_ref):
    hn = part_ref[0] + part_ref[1]
    x = h_ref[...] * hn
    y = jnp.dot(x, w_ref[...].T, preferred_element_type=jnp.float32)
    y = jnp.where(y > 0, y, 0.01 * y)
    nrm = jnp.sqrt(jnp.sum(y * y, axis=1, keepdims=True))
    h_out_ref[...] = y
    n_out_ref[...] = y / jnp.maximum(nrm, 1e-12)


def _dense_stage(part, h, W):
    d_out = W.shape[0]
    blk = 1000
    grid = (N_NODES // blk,)
    return pl.pallas_call(
        _dense_body,
        grid=grid,
        in_specs=[
            pl.BlockSpec((2, blk, D_IN), lambda i: (0, i, 0)),
            pl.BlockSpec((blk, D_IN), lambda i: (i, 0)),
            pl.BlockSpec((d_out, D_IN), lambda i: (0, 0)),
        ],
        out_specs=[
            pl.BlockSpec((blk, d_out), lambda i: (i, 0)),
            pl.BlockSpec((blk, d_out), lambda i: (i, 0)),
        ],
        out_shape=[
            jax.ShapeDtypeStruct((N_NODES, d_out), jnp.float32),
            jax.ShapeDtypeStruct((N_NODES, d_out), jnp.float32),
        ],
    )(part, h, W)


def kernel(node_ids, edge_index, edge_weight, entity_table, W0, W1):
    src = edge_index[0]
    dst = edge_index[1]
    w = edge_weight[:, 0]
    pad = E_PAD - N_EDGES
    src_p = jnp.pad(src, (0, pad)).reshape(N_TILES, N_CHUNKS, CHUNK)
    dst_p = jnp.pad(dst, (0, pad)).reshape(N_TILES, N_CHUNKS, CHUNK)
    w_p = jnp.pad(w, (0, pad)).reshape(N_TILES, N_CHUNKS, CHUNK)
    zeros = jnp.zeros((ROWS_PER_TILE, D_IN), jnp.float32)

    h = jnp.take(entity_table, node_ids, axis=0)
    cache = [h]
    for W in (W0, W1):
        part = _seg_call(h, src_p, dst_p, w_p, zeros)
        h, n = _dense_stage(part, h, W)
        cache.append(n)
    return jnp.concatenate(cache, axis=1)


# speedup vs baseline: 3.3669x; 1.2893x over previous
import functools

import jax
import jax.numpy as jnp
from jax import lax
from jax.experimental import pallas as pl
from jax.experimental.pallas import tpu as pltpu
from jax.experimental.pallas import tpu_sc as plsc

N_NODES = 10000
D_IN = 128
N_EDGES = 320000

N_TILES = 32
CHUNK = 64
NC = 160
E_PAD = N_TILES * NC * CHUNK
N_ACC = 10240
ROWS_PER_TILE = N_ACC // 16
NBUF = 4
NSLOT = 8


def _seg_body(h_hbm, src_hbm, dst_hbm, w_hbm, zeros_hbm, out_hbm,
              src_v, dst_v, w_v, rows_v, acc_sh,
              g_sem, s_sem, w_sem, si_sem, di_sem):
    cid = lax.axis_index("c")
    sid = lax.axis_index("s")
    wid = sid * 2 + cid

    def src_slice(c):
        return src_hbm.at[wid, c // 2, pl.ds((c % 2) * CHUNK, CHUNK)]

    def dst_slice(c):
        return dst_hbm.at[wid, c // 2, pl.ds((c % 2) * CHUNK, CHUNK)]

    pltpu.sync_copy(zeros_hbm, acc_sh.at[pl.ds(sid * ROWS_PER_TILE, ROWS_PER_TILE)])

    for k in range(NSLOT):
        pltpu.async_copy(src_slice(k), src_v.at[k], si_sem.at[k])
        pltpu.async_copy(dst_slice(k), dst_v.at[k], di_sem.at[k])
    for b in range(NBUF - 1):
        pltpu.make_async_copy(src_slice(b), src_v.at[b], si_sem.at[b]).wait()
        pltpu.async_copy(h_hbm.at[src_v.at[b]], rows_v.at[b], g_sem.at[b])
        pltpu.async_copy(w_hbm.at[wid, b], w_v.at[b], w_sem.at[b])
    plsc.subcore_barrier()

    def step(c, carry):
        b = c % NBUF
        sp = c % NSLOT
        cn = c + NBUF - 1
        bn = cn % NBUF

        pltpu.make_async_copy(h_hbm.at[src_v.at[sp]], rows_v.at[b],
                              g_sem.at[b]).wait()
        pltpu.make_async_copy(w_hbm.at[wid, c], w_v.at[b], w_sem.at[b]).wait()

        @pl.when(c + NSLOT < NC)
        def _():
            pltpu.async_copy(src_slice(c + NSLOT), src_v.at[sp], si_sem.at[sp])

        def edge_body(e, carry2):
            wv = w_v[b, pl.ds(e * 16, 16)]
            for j in range(D_IN // 16):
                sl = pl.ds(j * 16, 16)
                rows_v[b, e, sl] = rows_v[b, e, sl] * wv
            return carry2

        lax.fori_loop(0, CHUNK, edge_body, 0, unroll=2)

        pltpu.make_async_copy(dst_slice(c), dst_v.at[sp], di_sem.at[sp]).wait()
        pltpu.async_copy(rows_v.at[b], acc_sh.at[dst_v.at[sp]], s_sem.at[b],
                         add=True)

        @pl.when(cn < NC)
        def _():
            @pl.when(c >= 1)
            def _():
                pltpu.make_async_copy(
                    rows_v.at[bn], acc_sh.at[dst_v.at[(c - 1) % NSLOT]],
                    s_sem.at[bn]).wait()

                @pl.when(c + NSLOT - 1 < NC)
                def _():
                    pltpu.async_copy(dst_slice(c + NSLOT - 1),
                                     dst_v.at[(c - 1) % NSLOT],
                                     di_sem.at[(c - 1) % NSLOT])

            pltpu.make_async_copy(src_slice(cn), src_v.at[cn % NSLOT],
                                  si_sem.at[cn % NSLOT]).wait()
            pltpu.async_copy(h_hbm.at[src_v.at[cn % NSLOT]], rows_v.at[bn],
                             g_sem.at[bn])
            pltpu.async_copy(w_hbm.at[wid, cn], w_v.at[bn], w_sem.at[bn])

        return carry

    lax.fori_loop(0, NC, step, 0)

    for i in range(NC - NBUF, NC):
        pltpu.make_async_copy(rows_v.at[i % NBUF],
                              acc_sh.at[dst_v.at[i % NSLOT]],
                              s_sem.at[i % NBUF]).wait()
    plsc.subcore_barrier()

    sl = pl.ds(sid * ROWS_PER_TILE, ROWS_PER_TILE)
    pltpu.sync_copy(acc_sh.at[sl], out_hbm.at[cid, sl])


_seg_call = functools.partial(
    pl.kernel,
    out_type=jax.ShapeDtypeStruct((2, N_ACC, D_IN), jnp.float32),
    mesh=plsc.VectorSubcoreMesh(core_axis_name="c", subcore_axis_name="s"),
    compiler_params=pltpu.CompilerParams(needs_layout_passes=False),
    scratch_types=[
        pltpu.VMEM((NSLOT, CHUNK), jnp.int32),
        pltpu.VMEM((NSLOT, CHUNK), jnp.int32),
        pltpu.VMEM((NBUF, CHUNK * 16), jnp.float32),
        pltpu.VMEM((NBUF, CHUNK, D_IN), jnp.float32),
        pltpu.VMEM_SHARED((N_ACC, D_IN), jnp.float32),
        pltpu.SemaphoreType.DMA((NBUF,)),
        pltpu.SemaphoreType.DMA((NBUF,)),
        pltpu.SemaphoreType.DMA((NBUF,)),
        pltpu.SemaphoreType.DMA((NSLOT,)),
        pltpu.SemaphoreType.DMA((NSLOT,)),
    ],
)(_seg_body)


def _dense_body(part_ref, h_ref, w_ref, h_out_ref, n_out_ref):
    hn = part_ref[0] + part_ref[1]
    x = h_ref[...] * hn
    y = jnp.dot(x, w_ref[...].T, preferred_element_type=jnp.float32)
    y = jnp.where(y > 0, y, 0.01 * y)
    nrm = jnp.sqrt(jnp.sum(y * y, axis=1, keepdims=True))
    h_out_ref[...] = y
    n_out_ref[...] = y / jnp.maximum(nrm, 1e-12)


def _dense_stage(part, h, W):
    d_out = W.shape[0]
    blk = 1000
    grid = (N_NODES // blk,)
    return pl.pallas_call(
        _dense_body,
        grid=grid,
        in_specs=[
            pl.BlockSpec((2, blk, D_IN), lambda i: (0, i, 0)),
            pl.BlockSpec((blk, D_IN), lambda i: (i, 0)),
            pl.BlockSpec((d_out, D_IN), lambda i: (0, 0)),
        ],
        out_specs=[
            pl.BlockSpec((blk, d_out), lambda i: (i, 0)),
            pl.BlockSpec((blk, d_out), lambda i: (i, 0)),
        ],
        out_shape=[
            jax.ShapeDtypeStruct((N_NODES, d_out), jnp.float32),
            jax.ShapeDtypeStruct((N_NODES, d_out), jnp.float32),
        ],
    )(part, h, W)


def kernel(node_ids, edge_index, edge_weight, entity_table, W0, W1):
    src = edge_index[0]
    dst = edge_index[1]
    w = edge_weight[:, 0]
    pad = E_PAD - N_EDGES
    src_p = jnp.pad(src, (0, pad)).reshape(N_TILES, NC // 2, 2 * CHUNK)
    dst_p = jnp.pad(dst, (0, pad)).reshape(N_TILES, NC // 2, 2 * CHUNK)
    w_p = jnp.broadcast_to(
        jnp.pad(w, (0, pad)).reshape(N_TILES, NC, CHUNK)[..., None],
        (N_TILES, NC, CHUNK, 16),
    ).reshape(N_TILES, NC, CHUNK * 16)
    zeros = jnp.zeros((ROWS_PER_TILE, D_IN), jnp.float32)

    h = jnp.take(entity_table, node_ids, axis=0)
    cache = [h]
    for W in (W0, W1):
        part = _seg_call(h, src_p, dst_p, w_p, zeros)
        h, n = _dense_stage(part, h, W)
        cache.append(n)
    return jnp.concatenate(cache, axis=1)


# speedup vs baseline: 3.7680x; 1.1191x over previous
import functools

import jax
import jax.numpy as jnp
from jax import lax
from jax.experimental import pallas as pl
from jax.experimental.pallas import tpu as pltpu
from jax.experimental.pallas import tpu_sc as plsc

N_NODES = 10000
D_IN = 128
N_EDGES = 320000

CHUNK = 64
NCH = 5120
E_PAD = NCH * CHUNK
N_ACC = 10240
ROWS_PER_TILE = N_ACC // 16
NBUF = 4
NSLOT = 8
K0 = 248
K1 = 72


def _seg_body(h_hbm, src_hbm, dst_hbm, w_hbm, zeros_hbm, out_hbm,
              src_v, dst_v, w_v, rows_v, acc_sh,
              g_sem, s_sem, w_sem, si_sem, di_sem):
    cid = lax.axis_index("c")
    sid = lax.axis_index("s")
    lo = jnp.where(cid == 0, sid * K0, 16 * K0 + sid * K1)
    K = jnp.where(cid == 0, K0, K1)

    def src_slice(c):
        return src_hbm.at[c // 2, pl.ds((c % 2) * CHUNK, CHUNK)]

    def dst_slice(c):
        return dst_hbm.at[c // 2, pl.ds((c % 2) * CHUNK, CHUNK)]

    pltpu.sync_copy(zeros_hbm, acc_sh.at[pl.ds(sid * ROWS_PER_TILE, ROWS_PER_TILE)])

    for k in range(NSLOT):
        pltpu.async_copy(src_slice(lo + k), src_v.at[k], si_sem.at[k])
        pltpu.async_copy(dst_slice(lo + k), dst_v.at[k], di_sem.at[k])
    for b in range(NBUF - 1):
        pltpu.make_async_copy(src_slice(lo + b), src_v.at[b], si_sem.at[b]).wait()
        pltpu.async_copy(h_hbm.at[src_v.at[b]], rows_v.at[b], g_sem.at[b])
        pltpu.async_copy(w_hbm.at[lo + b], w_v.at[b], w_sem.at[b])
    plsc.subcore_barrier()

    def step(i, carry):
        c = lo + i
        b = i % NBUF
        sp = i % NSLOT
        inx = i + NBUF - 1
        bn = inx % NBUF

        pltpu.make_async_copy(h_hbm.at[src_v.at[sp]], rows_v.at[b],
                              g_sem.at[b]).wait()
        pltpu.make_async_copy(w_hbm.at[c], w_v.at[b], w_sem.at[b]).wait()

        @pl.when(i + NSLOT < K)
        def _():
            pltpu.async_copy(src_slice(c + NSLOT), src_v.at[sp], si_sem.at[sp])

        def edge_body(e, carry2):
            wv = w_v[b, pl.ds(e * 16, 16)]
            for j in range(D_IN // 16):
                sl = pl.ds(j * 16, 16)
                rows_v[b, e, sl] = rows_v[b, e, sl] * wv
            return carry2

        lax.fori_loop(0, CHUNK, edge_body, 0, unroll=2)

        pltpu.make_async_copy(dst_slice(c), dst_v.at[sp], di_sem.at[sp]).wait()
        pltpu.async_copy(rows_v.at[b], acc_sh.at[dst_v.at[sp]], s_sem.at[b],
                         add=True)

        @pl.when(inx < K)
        def _():
            @pl.when(i >= 1)
            def _():
                pltpu.make_async_copy(
                    rows_v.at[bn], acc_sh.at[dst_v.at[(i - 1) % NSLOT]],
                    s_sem.at[bn]).wait()

                @pl.when(i + NSLOT - 1 < K)
                def _():
                    pltpu.async_copy(dst_slice(c + NSLOT - 1),
                                     dst_v.at[(i - 1) % NSLOT],
                                     di_sem.at[(i - 1) % NSLOT])

            pltpu.make_async_copy(src_slice(lo + inx), src_v.at[inx % NSLOT],
                                  si_sem.at[inx % NSLOT]).wait()
            pltpu.async_copy(h_hbm.at[src_v.at[inx % NSLOT]], rows_v.at[bn],
                             g_sem.at[bn])
            pltpu.async_copy(w_hbm.at[lo + inx], w_v.at[bn], w_sem.at[bn])

        return carry

    lax.fori_loop(0, K, step, 0)

    for j in range(NBUF):
        i = K - NBUF + j
        pltpu.make_async_copy(rows_v.at[i % NBUF],
                              acc_sh.at[dst_v.at[i % NSLOT]],
                              s_sem.at[i % NBUF]).wait()
    plsc.subcore_barrier()

    sl = pl.ds(sid * ROWS_PER_TILE, ROWS_PER_TILE)
    pltpu.sync_copy(acc_sh.at[sl], out_hbm.at[cid, sl])


_seg_call = functools.partial(
    pl.kernel,
    out_type=jax.ShapeDtypeStruct((2, N_ACC, D_IN), jnp.float32),
    mesh=plsc.VectorSubcoreMesh(core_axis_name="c", subcore_axis_name="s"),
    compiler_params=pltpu.CompilerParams(needs_layout_passes=False),
    scratch_types=[
        pltpu.VMEM((NSLOT, CHUNK), jnp.int32),
        pltpu.VMEM((NSLOT, CHUNK), jnp.int32),
        pltpu.VMEM((NBUF, CHUNK * 16), jnp.float32),
        pltpu.VMEM((NBUF, CHUNK, D_IN), jnp.float32),
        pltpu.VMEM_SHARED((N_ACC, D_IN), jnp.float32),
        pltpu.SemaphoreType.DMA((NBUF,)),
        pltpu.SemaphoreType.DMA((NBUF,)),
        pltpu.SemaphoreType.DMA((NBUF,)),
        pltpu.SemaphoreType.DMA((NSLOT,)),
        pltpu.SemaphoreType.DMA((NSLOT,)),
    ],
)(_seg_body)


def _expand_body(w_ref, out_ref):
    sel = (lax.broadcasted_iota(jnp.int32, (CHUNK, CHUNK * 16), 1) // 16
           == lax.broadcasted_iota(jnp.int32, (CHUNK, CHUNK * 16), 0)
           ).astype(jnp.float32)
    out_ref[...] = jnp.dot(w_ref[...], sel, preferred_element_type=jnp.float32)


def _expand_w(w_pad):
    blk = 512
    return pl.pallas_call(
        _expand_body,
        grid=(NCH // blk,),
        in_specs=[pl.BlockSpec((blk, CHUNK), lambda i: (i, 0))],
        out_specs=pl.BlockSpec((blk, CHUNK * 16), lambda i: (i, 0)),
        out_shape=jax.ShapeDtypeStruct((NCH, CHUNK * 16), jnp.float32),
    )(w_pad)


def _dense_body(part_ref, h_ref, w_ref, h_out_ref, n_out_ref):
    hn = part_ref[0] + part_ref[1]
    x = h_ref[...] * hn
    y = jnp.dot(x, w_ref[...].T, preferred_element_type=jnp.float32)
    y = jnp.where(y > 0, y, 0.01 * y)
    nrm = jnp.sqrt(jnp.sum(y * y, axis=1, keepdims=True))
    h_out_ref[...] = y
    n_out_ref[...] = y / jnp.maximum(nrm, 1e-12)


def _dense_stage(part, h, W):
    d_out = W.shape[0]
    blk = 1000
    grid = (N_NODES // blk,)
    return pl.pallas_call(
        _dense_body,
        grid=grid,
        in_specs=[
            pl.BlockSpec((2, blk, D_IN), lambda i: (0, i, 0)),
            pl.BlockSpec((blk, D_IN), lambda i: (i, 0)),
            pl.BlockSpec((d_out, D_IN), lambda i: (0, 0)),
        ],
        out_specs=[
            pl.BlockSpec((blk, d_out), lambda i: (i, 0)),
            pl.BlockSpec((blk, d_out), lambda i: (i, 0)),
        ],
        out_shape=[
            jax.ShapeDtypeStruct((N_NODES, d_out), jnp.float32),
            jax.ShapeDtypeStruct((N_NODES, d_out), jnp.float32),
        ],
    )(part, h, W)


def kernel(node_ids, edge_index, edge_weight, entity_table, W0, W1):
    src = edge_index[0]
    dst = edge_index[1]
    w = edge_weight[:, 0]
    pad = E_PAD - N_EDGES
    src_p = jnp.pad(src, (0, pad)).reshape(NCH // 2, 2 * CHUNK)
    dst_p = jnp.pad(dst, (0, pad)).reshape(NCH // 2, 2 * CHUNK)
    w_p = _expand_w(jnp.pad(w, (0, pad)).reshape(NCH, CHUNK))
    zeros = jnp.zeros((ROWS_PER_TILE, D_IN), jnp.float32)

    h = jnp.take(entity_table, node_ids, axis=0)
    cache = [h]
    for W in (W0, W1):
        part = _seg_call(h, src_p, dst_p, w_p, zeros)
        h, n = _dense_stage(part, h, W)
        cache.append(n)
    return jnp.concatenate(cache, axis=1)
